# Initial kernel scaffold; baseline (speedup 1.0000x reference)
import jax
import jax.numpy as jnp
from jax.experimental import pallas as pl


def kernel(x, edge_index, W, b):
    raise NotImplementedError("write your pallas kernel here")


# speedup vs baseline: 15.6442x; 15.6442x over previous
import functools

import jax
import jax.numpy as jnp
from jax import lax
from jax.experimental import pallas as pl
from jax.experimental.pallas import tpu as pltpu
from jax.experimental.pallas import tpu_sc as plsc

N, E, D = 10000, 320000, 128
L = 16
NC, NS = 2, 16
NW = NC * NS
NPAD = 10240
EPAD = 327680
RW = NPAD // NW
EH = EPAD // NS
ET = EPAD // NW
CH = 64
NCH = ET // CH
NB = 2
RS = 4
RPT = NPAD // NS
HR = NPAD // L
HC = 128
XC = 32

_mesh = plsc.VectorSubcoreMesh(core_axis_name="c", subcore_axis_name="s")


@functools.partial(
    pl.kernel,
    out_type=(
        jax.ShapeDtypeStruct((NPAD, D), jnp.float32),
        jax.ShapeDtypeStruct((NPAD,), jnp.float32),
    ),
    mesh=_mesh,
    scratch_types=[
        pltpu.VMEM((EH,), jnp.int32),
        pltpu.VMEM((NPAD,), jnp.float32),
        pltpu.VMEM((RPT,), jnp.float32),
        pltpu.VMEM((RPT,), jnp.float32),
        pltpu.VMEM((RW,), jnp.float32),
        pltpu.VMEM((XC, D), jnp.float32),
        pltpu.VMEM_SHARED((NS, 1, NPAD), jnp.float32),
    ],
    compiler_params=pltpu.CompilerParams(needs_layout_passes=False),
)
def _k1(row_hbm, x_hbm, xt_hbm, dis_hbm,
        rows_v, hist_v, deg_v, tmp_v, dis_v, xr_v, hist_sh):
    c = lax.axis_index("c")
    s = lax.axis_index("s")
    w = c * NS + s

    def zrow(i, _):
        hist_v[pl.ds(i * L, L)] = jnp.zeros((L,), jnp.float32)
        return 0
    lax.fori_loop(0, HR, zrow, 0)

    pltpu.sync_copy(row_hbm.at[pl.ds(s * EH, EH)], rows_v)

    ones = jnp.ones((L,), jnp.float32)

    def hbody(i, _):
        idx = rows_v[pl.ds(i * L, L)]
        plsc.addupdate_scatter(hist_v, [idx], ones)
        return 0
    lax.fori_loop(0, EH // L, hbody, 0)

    pltpu.sync_copy(hist_v, hist_sh.at[s, 0])

    plsc.subcore_barrier()

    def zdeg(r, _):
        deg_v[pl.ds(r * L, L)] = jnp.zeros((L,), jnp.float32)
        return 0
    lax.fori_loop(0, RPT // L, zdeg, 0)
    for t in range(NS):
        pltpu.sync_copy(hist_sh.at[t, 0, pl.ds(s * RPT, RPT)], tmp_v)

        def accum(r, _):
            deg_v[pl.ds(r * L, L)] = (deg_v[pl.ds(r * L, L)]
                                      + tmp_v[pl.ds(r * L, L)])
            return 0
        lax.fori_loop(0, RPT // L, accum, 0)

    half = c * RW
    nbase = s * RPT + half

    def dbody(r, _):
        dg = deg_v[pl.ds(half + r * L, L)] + 1.0
        yi = jnp.int32(0x5F3759DF) - lax.shift_right_logical(
            plsc.bitcast(dg, jnp.int32), 1)
        y = plsc.bitcast(yi, jnp.float32)
        h = dg * 0.5
        y = y * (1.5 - h * y * y)
        y = y * (1.5 - h * y * y)
        y = y * (1.5 - h * y * y)
        dis_v[pl.ds(r * L, L)] = y
        return 0
    lax.fori_loop(0, RW // L, dbody, 0)

    pltpu.sync_copy(dis_v, dis_hbm.at[pl.ds(nbase, RW)])

    for t in range(RW // XC):
        base = nbase + t * XC
        pltpu.sync_copy(x_hbm.at[pl.ds(base, XC)], xr_v)

        def sbody(r16, _):
            d16 = dis_v[pl.ds(t * XC + r16 * L, L)]
            for lane in range(L):
                r = r16 * L + lane
                dd = d16[lane]
                for g in range(D // L):
                    xr_v[r, pl.ds(g * L, L)] = xr_v[r, pl.ds(g * L, L)] * dd
            return 0
        lax.fori_loop(0, XC // L, sbody, 0)

        pltpu.sync_copy(xr_v, xt_hbm.at[pl.ds(base, XC)])


@functools.partial(
    pl.kernel,
    out_type=jax.ShapeDtypeStruct((NC, NPAD, D), jnp.float32),
    mesh=_mesh,
    scratch_types=[
        pltpu.VMEM((NCH, CH), jnp.int32),
        pltpu.VMEM((RS, 2, CH), jnp.int32),
        pltpu.VMEM((NB, CH, D), jnp.float32),
        pltpu.SemaphoreType.DMA,
        pltpu.SemaphoreType.DMA,
        pltpu.SemaphoreType.DMA,
        pltpu.SemaphoreType.DMA,
        pltpu.VMEM_SHARED((NPAD, D), jnp.float32),
    ],
    compiler_params=pltpu.CompilerParams(needs_layout_passes=False),
)
def _k2(ids_hbm, xt_hbm, acc_hbm, pids, ring, bufv, g0, g1, s0, s1, acc_sh):
    c = lax.axis_index("c")
    s = lax.axis_index("s")
    w = c * NS + s
    bufs = tuple(bufv.at[u] for u in range(NB))
    gsem = (g0, g1)
    ssem = (s0, s1)

    pltpu.sync_copy(ids_hbm.at[w], pids)

    def unpack(j):
        q = lax.rem(j, RS)
        for g in range(CH // L):
            v = pids[j, pl.ds(g * L, L)]
            ring[q, 0, pl.ds(g * L, L)] = lax.shift_right_logical(v, 14)
            ring[q, 1, pl.ds(g * L, L)] = lax.bitwise_and(v, 16383)

    @pl.when(c == 0)
    def _():
        for t in range(RPT // CH):
            pltpu.sync_copy(xt_hbm.at[pl.ds(s * RPT + t * CH, CH)], bufs[0])
            pltpu.sync_copy(bufs[0], acc_sh.at[pl.ds(s * RPT + t * CH, CH)])

    @pl.when(c == 1)
    def _():
        def zr(r, _):
            for g in range(D // L):
                bufv[0, r, pl.ds(g * L, L)] = jnp.zeros((L,), jnp.float32)
            return 0
        lax.fori_loop(0, CH, zr, 0)
        for t in range(RPT // CH):
            pltpu.sync_copy(bufs[0], acc_sh.at[pl.ds(s * RPT + t * CH, CH)])

    plsc.subcore_barrier()

    for u in range(NB):
        unpack(jnp.int32(u))
        pltpu.async_copy(xt_hbm.at[ring.at[u, 1]], bufs[u], gsem[u])

    def step(t, _):
        for u in range(NB):
            j = t * NB + u
            q = lax.rem(j, RS)
            pltpu.make_async_copy(
                xt_hbm.at[ring.at[q, 1]], bufs[u], gsem[u]).wait()
            pltpu.async_copy(bufs[u], acc_sh.at[ring.at[q, 0]], ssem[u],
                             add=True)

            @pl.when(t < NCH // NB - 1)
            def _():
                pltpu.make_async_copy(
                    bufs[u], acc_sh.at[ring.at[q, 0]], ssem[u]).wait()
                jn = j + NB
                qn = lax.rem(jn, RS)
                unpack(jn)
                pltpu.async_copy(
                    xt_hbm.at[ring.at[qn, 1]], bufs[u], gsem[u])
        return 0
    lax.fori_loop(0, NCH // NB, step, 0)

    for u in range(NB):
        j = NCH - NB + u
        q = lax.rem(jnp.int32(j), RS)
        pltpu.make_async_copy(
            bufs[u], acc_sh.at[ring.at[q, 0]], ssem[u]).wait()

    plsc.subcore_barrier()

    for t in range(RPT // CH):
        pltpu.sync_copy(acc_sh.at[pl.ds(s * RPT + t * CH, CH)], bufs[0])
        pltpu.sync_copy(bufs[0], acc_hbm.at[c, pl.ds(s * RPT + t * CH, CH)])


BLK = 1280


def _k3_body(acc_ref, dis_ref, w_ref, b_ref, o_ref):
    sd = (acc_ref[0] + acc_ref[1]) * dis_ref[...]
    o_ref[...] = lax.dot_general(
        sd, w_ref[...], (((1,), (1,)), ((), ())),
        preferred_element_type=jnp.float32) + b_ref[...]


_k3 = pl.pallas_call(
    _k3_body,
    grid=(NPAD // BLK,),
    in_specs=[
        pl.BlockSpec((NC, BLK, D), lambda i: (0, i, 0)),
        pl.BlockSpec((BLK, 1), lambda i: (i, 0)),
        pl.BlockSpec((D, D), lambda i: (0, 0)),
        pl.BlockSpec((1, D), lambda i: (0, 0)),
    ],
    out_specs=pl.BlockSpec((BLK, D), lambda i: (i, 0)),
    out_shape=jax.ShapeDtypeStruct((NPAD, D), jnp.float32),
)


def kernel(x, edge_index, W, b):
    row = edge_index[0].astype(jnp.int32)
    col = edge_index[1].astype(jnp.int32)
    pad_e = jnp.full((EPAD - E,), N, jnp.int32)
    row_p = jnp.concatenate([row, pad_e])
    col_p = jnp.concatenate([col, pad_e])
    x_p = jnp.pad(x, ((0, NPAD - N), (0, 0)))

    xt, dis = _k1(row_p, x_p)
    packed = (row_p * 16384 + col_p).reshape(NW, NCH, CH)
    acc = _k2(packed, xt)
    out = _k3(acc, dis.reshape(NPAD, 1), W, b.reshape(1, D))
    return out[:N]
